# Initial kernel scaffold; baseline (speedup 1.0000x reference)
import jax
import jax.numpy as jnp
from jax.experimental import pallas as pl


def kernel(a, b, threshold, h0):
    raise NotImplementedError("write your pallas kernel here")


# speedup vs baseline: 50.0913x; 50.0913x over previous
import jax
import jax.numpy as jnp
from jax.experimental import pallas as pl
from jax.experimental.pallas import tpu as pltpu

BT = 128
BD = 256


def _scan_body(a_ref, b_ref, thr_ref, h0_ref, h_out_ref, s_out_ref, h_scr):
    t_blk = pl.program_id(1)

    @pl.when(t_blk == 0)
    def _():
        h_scr[...] = h0_ref[...]

    thr = thr_ref[...]

    def step(i, h):
        h_pre = a_ref[:, i, :] * h + b_ref[:, i, :]
        spike = h_pre > thr
        h_post = jnp.where(spike, 0.0, h_pre)
        h_out_ref[:, i, :] = h_post
        s_out_ref[:, i, :] = spike.astype(jnp.float32)
        return h_post

    h_scr[...] = jax.lax.fori_loop(0, BT, step, h_scr[...], unroll=8)


def kernel(a, b, threshold, h0):
    B, T, D = a.shape
    thr2d = threshold.reshape(1, D)

    grid = (D // BD, T // BT)
    blk_btd = pl.BlockSpec((B, BT, BD), lambda d, t: (0, t, d))

    h_post, s = pl.pallas_call(
        _scan_body,
        out_shape=(
            jax.ShapeDtypeStruct((B, T, D), jnp.float32),
            jax.ShapeDtypeStruct((B, T, D), jnp.float32),
        ),
        grid=grid,
        in_specs=[
            blk_btd,
            blk_btd,
            pl.BlockSpec((1, BD), lambda d, t: (0, d)),
            pl.BlockSpec((B, BD), lambda d, t: (0, d)),
        ],
        out_specs=(blk_btd, blk_btd),
        scratch_shapes=[pltpu.VMEM((B, BD), jnp.float32)],
        compiler_params=pltpu.CompilerParams(
            dimension_semantics=("parallel", "arbitrary"),
        ),
        name="lif_scan",
    )(a, b, thr2d, h0)
    return h_post, s


# speedup vs baseline: 67.2362x; 1.3423x over previous
import jax
import jax.numpy as jnp
from jax.experimental import pallas as pl
from jax.experimental.pallas import tpu as pltpu

BT = 32
BD = 512


def _scan_body(a_ref, b_ref, thr_ref, h0_ref, h_out_ref, s_out_ref, h_scr):
    t_blk = pl.program_id(1)

    @pl.when(t_blk == 0)
    def _():
        h_scr[...] = h0_ref[...]

    thr = thr_ref[...]
    h = h_scr[...]
    for i in range(BT):
        h_pre = a_ref[:, i, :] * h + b_ref[:, i, :]
        spike = h_pre > thr
        h = jnp.where(spike, 0.0, h_pre)
        h_out_ref[:, i, :] = h
        s_out_ref[:, i, :] = jnp.where(spike, 1.0, 0.0)
    h_scr[...] = h


def kernel(a, b, threshold, h0):
    B, T, D = a.shape
    thr2d = threshold.reshape(1, D)

    grid = (D // BD, T // BT)
    blk_btd = pl.BlockSpec((B, BT, BD), lambda d, t: (0, t, d))

    h_post, s = pl.pallas_call(
        _scan_body,
        out_shape=(
            jax.ShapeDtypeStruct((B, T, D), jnp.float32),
            jax.ShapeDtypeStruct((B, T, D), jnp.float32),
        ),
        grid=grid,
        in_specs=[
            blk_btd,
            blk_btd,
            pl.BlockSpec((1, BD), lambda d, t: (0, d)),
            pl.BlockSpec((B, BD), lambda d, t: (0, d)),
        ],
        out_specs=(blk_btd, blk_btd),
        scratch_shapes=[pltpu.VMEM((B, BD), jnp.float32)],
        compiler_params=pltpu.CompilerParams(
            dimension_semantics=("parallel", "arbitrary"),
        ),
        name="lif_scan",
    )(a, b, thr2d, h0)
    return h_post, s


# speedup vs baseline: 68.8981x; 1.0247x over previous
import jax
import jax.numpy as jnp
from jax.experimental import pallas as pl
from jax.experimental.pallas import tpu as pltpu

BT = 32
BB = 8


def _scan_body(a_ref, b_ref, thr_ref, h0_ref, h_out_ref, s_out_ref, h_scr):
    t_blk = pl.program_id(1)

    @pl.when(t_blk == 0)
    def _():
        h_scr[...] = h0_ref[...]

    thr = thr_ref[...]
    h = h_scr[...]
    for i in range(BT):
        h_pre = a_ref[:, i, :] * h + b_ref[:, i, :]
        spike = h_pre > thr
        h = jnp.where(spike, 0.0, h_pre)
        h_out_ref[:, i, :] = h
        s_out_ref[:, i, :] = jnp.where(spike, 1.0, 0.0)
    h_scr[...] = h


def kernel(a, b, threshold, h0):
    B, T, D = a.shape
    thr2d = threshold.reshape(1, D)

    grid = (B // BB, T // BT)
    blk_btd = pl.BlockSpec((BB, BT, D), lambda p, t: (p, t, 0))

    h_post, s = pl.pallas_call(
        _scan_body,
        out_shape=(
            jax.ShapeDtypeStruct((B, T, D), jnp.float32),
            jax.ShapeDtypeStruct((B, T, D), jnp.float32),
        ),
        grid=grid,
        in_specs=[
            blk_btd,
            blk_btd,
            pl.BlockSpec((1, D), lambda p, t: (0, 0)),
            pl.BlockSpec((BB, D), lambda p, t: (p, 0)),
        ],
        out_specs=(blk_btd, blk_btd),
        scratch_shapes=[pltpu.VMEM((BB, D), jnp.float32)],
        compiler_params=pltpu.CompilerParams(
            dimension_semantics=("parallel", "arbitrary"),
        ),
        name="lif_scan",
    )(a, b, thr2d, h0)
    return h_post, s


# speedup vs baseline: 79.8380x; 1.1588x over previous
import jax
import jax.numpy as jnp
from jax.experimental import pallas as pl
from jax.experimental.pallas import tpu as pltpu

BT = 64
BB = 8


def _scan_body(a_ref, b_ref, thr_ref, h0_ref, h_out_ref, s_out_ref, h_scr):
    t_blk = pl.program_id(1)

    @pl.when(t_blk == 0)
    def _():
        h_scr[...] = h0_ref[...]

    thr = thr_ref[...]
    h = h_scr[...]
    for i in range(BT):
        h_pre = a_ref[:, i, :] * h + b_ref[:, i, :]
        spike = h_pre > thr
        h = jnp.where(spike, 0.0, h_pre)
        h_out_ref[:, i, :] = h
        s_out_ref[:, i, :] = jnp.where(spike, 1.0, 0.0)
    h_scr[...] = h


def kernel(a, b, threshold, h0):
    B, T, D = a.shape
    thr2d = threshold.reshape(1, D)

    grid = (B // BB, T // BT)
    blk_btd = pl.BlockSpec((BB, BT, D), lambda p, t: (p, t, 0))

    h_post, s = pl.pallas_call(
        _scan_body,
        out_shape=(
            jax.ShapeDtypeStruct((B, T, D), jnp.float32),
            jax.ShapeDtypeStruct((B, T, D), jnp.float32),
        ),
        grid=grid,
        in_specs=[
            blk_btd,
            blk_btd,
            pl.BlockSpec((1, D), lambda p, t: (0, 0)),
            pl.BlockSpec((BB, D), lambda p, t: (p, 0)),
        ],
        out_specs=(blk_btd, blk_btd),
        scratch_shapes=[pltpu.VMEM((BB, D), jnp.float32)],
        compiler_params=pltpu.CompilerParams(
            dimension_semantics=("parallel", "arbitrary"),
        ),
        name="lif_scan",
    )(a, b, thr2d, h0)
    return h_post, s


# speedup vs baseline: 83.7194x; 1.0486x over previous
import jax
import jax.numpy as jnp
from jax.experimental import pallas as pl
from jax.experimental.pallas import tpu as pltpu

BT = 128
BB = 8


def _scan_body(a_ref, b_ref, thr_ref, h0_ref, h_out_ref, s_out_ref, h_scr):
    t_blk = pl.program_id(1)

    @pl.when(t_blk == 0)
    def _():
        h_scr[...] = h0_ref[...]

    thr = thr_ref[...]
    h = h_scr[...]
    for i in range(BT):
        h_pre = a_ref[:, i, :] * h + b_ref[:, i, :]
        spike = h_pre > thr
        h = jnp.where(spike, 0.0, h_pre)
        h_out_ref[:, i, :] = h
        s_out_ref[:, i, :] = jnp.where(spike, 1.0, 0.0)
    h_scr[...] = h


def kernel(a, b, threshold, h0):
    B, T, D = a.shape
    thr2d = threshold.reshape(1, D)

    grid = (B // BB, T // BT)
    blk_btd = pl.BlockSpec((BB, BT, D), lambda p, t: (p, t, 0))

    h_post, s = pl.pallas_call(
        _scan_body,
        out_shape=(
            jax.ShapeDtypeStruct((B, T, D), jnp.float32),
            jax.ShapeDtypeStruct((B, T, D), jnp.float32),
        ),
        grid=grid,
        in_specs=[
            blk_btd,
            blk_btd,
            pl.BlockSpec((1, D), lambda p, t: (0, 0)),
            pl.BlockSpec((BB, D), lambda p, t: (p, 0)),
        ],
        out_specs=(blk_btd, blk_btd),
        scratch_shapes=[pltpu.VMEM((BB, D), jnp.float32)],
        compiler_params=pltpu.CompilerParams(
            dimension_semantics=("parallel", "arbitrary"),
        ),
        name="lif_scan",
    )(a, b, thr2d, h0)
    return h_post, s
